# baseline (speedup 1.0000x reference)
import functools

import jax
import jax.numpy as jnp
from jax import lax
from jax.experimental import pallas as pl
from jax.experimental.pallas import tpu as pltpu
from jax.experimental.pallas import tpu_sc as plsc

_NUM_CORES = 2
_NUM_SUBCORES = 16
_LANES = 16
_IDX_CHUNK = 128


@functools.lru_cache(maxsize=None)
def _make_lookup(batch, rows, n_chunks, chunk_size, vocab):
    nw = _NUM_CORES * _NUM_SUBCORES
    bw = batch // nw
    fl = bw * n_chunks
    n_xdma = bw // _IDX_CHUNK
    n_tdma = fl // _IDX_CHUNK
    hrows = vocab * n_chunks // _LANES
    assert batch % (nw * _IDX_CHUNK) == 0
    assert n_chunks & (n_chunks - 1) == 0 and _LANES % n_chunks == 0
    assert (vocab * n_chunks) % _LANES == 0
    log2c = n_chunks.bit_length() - 1
    xpr = _LANES // n_chunks
    log2xpr = xpr.bit_length() - 1
    vpr = _IDX_CHUNK // _LANES

    mesh = plsc.VectorSubcoreMesh(
        core_axis_name="c", subcore_axis_name="s",
        num_cores=_NUM_CORES, num_subcores=_NUM_SUBCORES)

    @functools.partial(
        pl.kernel,
        out_type=jax.ShapeDtypeStruct((batch * n_chunks, chunk_size),
                                      jnp.float32),
        mesh=mesh,
        compiler_params=pltpu.CompilerParams(
            needs_layout_passes=False, use_tc_tiling_on_sc=False),
        scratch_types=[
            pltpu.VMEM((bw,), jnp.int32),
            pltpu.VMEM((n_xdma, _IDX_CHUNK), jnp.int32),
            pltpu.VMEM((bw, _LANES), jnp.int32),
            pltpu.VMEM((bw, _LANES), jnp.int32),
            pltpu.VMEM((n_tdma, _IDX_CHUNK), jnp.int32),
            pltpu.VMEM((n_tdma, _IDX_CHUNK), jnp.int32),
            pltpu.VMEM((fl, chunk_size), jnp.float32),
            pltpu.VMEM((fl, chunk_size), jnp.float32),
            pltpu.SemaphoreType.DMA,
            pltpu.SemaphoreType.DMA,
        ],
    )
    def lookup(x_hbm, t0_hbm, t1_hbm, h0_hbm, h1_hbm, out_hbm,
               x_v, xq, g0, g1, f0, f1, a0, a1, s0, s1):
        wid = lax.axis_index("s") * _NUM_CORES + lax.axis_index("c")
        pltpu.sync_copy(x_hbm.at[pl.ds(wid * bw, bw)], x_v)

        lane = lax.iota(jnp.int32, _LANES)

        def xq_body(k, carry):
            for u in range(vpr):
                i = k * vpr + u
                xq[k, pl.ds(u * _LANES, _LANES)] = (
                    x_v[pl.ds(i * _LANES, _LANES)] >> log2xpr)
            return carry
        lax.fori_loop(0, n_xdma, xq_body, 0)

        hcopies = []
        for j in range(n_xdma):
            hcopies.append(pltpu.async_copy(
                h0_hbm.at[xq.at[j]],
                g0.at[pl.ds(j * _IDX_CHUNK, _IDX_CHUNK)], s0))
            hcopies.append(pltpu.async_copy(
                h1_hbm.at[xq.at[j]],
                g1.at[pl.ds(j * _IDX_CHUNK, _IDX_CHUNK)], s1))
        for c in hcopies:
            c.wait()

        col = lane & (n_chunks - 1)

        def flat_body(j, carry):
            for u in range(vpr):
                p = j * _IDX_CHUNK + u * _LANES + lane
                b = p >> log2c
                xb = plsc.load_gather(x_v, [b])
                coff = ((xb & (xpr - 1)) << log2c) + col
                hv0 = plsc.load_gather(g0, [b, coff])
                hv1 = plsc.load_gather(g1, [b, coff])
                f0[j, pl.ds(u * _LANES, _LANES)] = hv0 * n_chunks + col
                f1[j, pl.ds(u * _LANES, _LANES)] = hv1 * n_chunks + col
            return carry
        lax.fori_loop(0, n_tdma, flat_body, 0)

        gcopies = []
        for j in range(n_tdma):
            gcopies.append(pltpu.async_copy(
                t0_hbm.at[f0.at[j]],
                a0.at[pl.ds(j * _IDX_CHUNK, _IDX_CHUNK)], s0))
            gcopies.append(pltpu.async_copy(
                t1_hbm.at[f1.at[j]],
                a1.at[pl.ds(j * _IDX_CHUNK, _IDX_CHUNK)], s1))
        for c in gcopies:
            c.wait()

        def add_body(k, carry):
            a0[k, :] = a0[k, :] + a1[k, :]
            return carry
        lax.fori_loop(0, fl, add_body, 0)

        pltpu.sync_copy(a0, out_hbm.at[pl.ds(wid * fl, fl)])

    return lookup


def kernel(x, table0, table1, h0, h1):
    rows, n_chunks, chunk_size = table0.shape
    vocab = h0.shape[0]
    batch = x.shape[0]
    lookup = _make_lookup(batch, rows, n_chunks, chunk_size, vocab)
    out = lookup(
        x.astype(jnp.int32),
        table0.reshape(rows * n_chunks, chunk_size),
        table1.reshape(rows * n_chunks, chunk_size),
        h0.reshape(vocab * n_chunks // _LANES, _LANES),
        h1.reshape(vocab * n_chunks // _LANES, _LANES))
    return out.reshape(batch, n_chunks * chunk_size)
